# initial kernel scaffold (unmeasured)
import jax
import jax.numpy as jnp
from jax import lax
from jax.experimental import pallas as pl
from jax.experimental.pallas import tpu as pltpu

N_DEV = 16
M, N = 4096, 2048
CH = M // N_DEV
N_STEPS = 2 * (N_DEV - 1)


def kernel(x, w_mat, scale_x, scale_w):
    def body(x_ref, w_ref, sx_ref, sw_ref, out_ref,
             comm_ref, send_sems, recv_sems, credit_sem):
        my = lax.axis_index("i")
        left = lax.rem(my + N_DEV - 1, N_DEV)
        right = lax.rem(my + 1, N_DEV)

        barrier_sem = pltpu.get_barrier_semaphore()
        for nbr in (left, right):
            pl.semaphore_signal(barrier_sem, inc=1, device_id=(nbr,),
                                device_id_type=pl.DeviceIdType.MESH)
        pl.semaphore_wait(barrier_sem, 2)

        def gemm(c):
            return jnp.dot(
                x_ref[pl.ds(c * CH, CH), :].astype(jnp.bfloat16),
                w_ref[...].astype(jnp.bfloat16),
                preferred_element_type=jnp.float32,
            )

        scale = sx_ref[0] * sw_ref[0]

        comm_ref[0, :, :] = gemm(my)

        for s in range(N_STEPS):
            send_slot = s % 2
            recv_slot = (s + 1) % 2
            if s >= 2:
                pl.semaphore_wait(credit_sem, 1)
            rdma = pltpu.make_async_remote_copy(
                src_ref=comm_ref.at[send_slot],
                dst_ref=comm_ref.at[recv_slot],
                send_sem=send_sems.at[send_slot],
                recv_sem=recv_sems.at[recv_slot],
                device_id=(right,),
                device_id_type=pl.DeviceIdType.MESH,
            )
            rdma.start()
            rdma.wait()

            if s < N_DEV - 1:
                c = lax.rem(my - s - 1 + N_DEV, N_DEV)
                acc = comm_ref[recv_slot, :, :] + gemm(c)
                if s == N_DEV - 2:
                    acc = jnp.maximum(acc * scale, 0.0)
                    out_ref[pl.ds(c * CH, CH), :] = acc
                comm_ref[recv_slot, :, :] = acc
            else:
                c = lax.rem(my - (s - N_DEV + 1) + N_DEV, N_DEV)
                out_ref[pl.ds(c * CH, CH), :] = comm_ref[recv_slot, :, :]

            if s <= N_STEPS - 3:
                pl.semaphore_signal(credit_sem, inc=1, device_id=(left,),
                                    device_id_type=pl.DeviceIdType.MESH)

    return pl.pallas_call(
        body,
        out_shape=jax.ShapeDtypeStruct((M, N), jnp.float32),
        in_specs=[
            pl.BlockSpec(memory_space=pltpu.VMEM),
            pl.BlockSpec(memory_space=pltpu.VMEM),
            pl.BlockSpec(memory_space=pltpu.SMEM),
            pl.BlockSpec(memory_space=pltpu.SMEM),
        ],
        out_specs=pl.BlockSpec(memory_space=pltpu.VMEM),
        scratch_shapes=[
            pltpu.VMEM((2, CH, N), jnp.float32),
            pltpu.SemaphoreType.DMA((2,)),
            pltpu.SemaphoreType.DMA((2,)),
            pltpu.SemaphoreType.REGULAR,
        ],
        compiler_params=pltpu.CompilerParams(collective_id=0),
    )(x, w_mat, scale_x, scale_w)


# baseline (device time: 779210 ns/iter reference)
import jax
import jax.numpy as jnp
from jax import lax
from jax.experimental import pallas as pl
from jax.experimental.pallas import tpu as pltpu

N_DEV = 16
M, N = 4096, 2048
CH = M // N_DEV
N_STEPS = 2 * (N_DEV - 1)


def kernel(x, w_mat, scale_x, scale_w):
    def body(x_ref, w_ref, sx_ref, sw_ref, out_ref,
             comm_ref, send_sems, recv_sems, credit_sem):
        my = lax.axis_index("i")
        left = lax.rem(my + N_DEV - 1, N_DEV)
        right = lax.rem(my + 1, N_DEV)

        barrier_sem = pltpu.get_barrier_semaphore()
        for nbr in (left, right):
            pl.semaphore_signal(barrier_sem, inc=1, device_id=(nbr,),
                                device_id_type=pl.DeviceIdType.MESH)
        pl.semaphore_wait(barrier_sem, 2)

        def gemm(c):
            return jnp.dot(
                x_ref[pl.ds(c * CH, CH), :].astype(jnp.bfloat16),
                w_ref[...].astype(jnp.bfloat16),
                preferred_element_type=jnp.float32,
            )

        scale = sx_ref[0] * sw_ref[0]

        comm_ref[0, :, :] = gemm(my)

        for s in range(N_STEPS):
            send_slot = s % 2
            recv_slot = (s + 1) % 2
            if s >= 2:
                pl.semaphore_wait(credit_sem, 1)
            rdma = pltpu.make_async_remote_copy(
                src_ref=comm_ref.at[send_slot],
                dst_ref=comm_ref.at[recv_slot],
                send_sem=send_sems.at[send_slot],
                recv_sem=recv_sems.at[recv_slot],
                device_id=(right,),
                device_id_type=pl.DeviceIdType.MESH,
            )
            rdma.start()
            rdma.wait()

            if s < N_DEV - 1:
                c = lax.rem(my - s - 1 + N_DEV, N_DEV)
                acc = comm_ref[recv_slot, :, :] + gemm(c)
                if s == N_DEV - 2:
                    acc = jnp.maximum(acc * scale, 0.0)
                    out_ref[pl.ds(c * CH, CH), :] = acc
                comm_ref[recv_slot, :, :] = acc
            else:
                c = lax.rem(my - (s - N_DEV + 1) + N_DEV, N_DEV)
                out_ref[pl.ds(c * CH, CH), :] = comm_ref[recv_slot, :, :]

            if s <= N_STEPS - 3:
                pl.semaphore_signal(credit_sem, inc=1, device_id=(left,),
                                    device_id_type=pl.DeviceIdType.MESH)

    return pl.pallas_call(
        body,
        out_shape=jax.ShapeDtypeStruct((M, N), jnp.float32),
        in_specs=[
            pl.BlockSpec(memory_space=pltpu.VMEM),
            pl.BlockSpec(memory_space=pltpu.VMEM),
            pl.BlockSpec(memory_space=pltpu.SMEM),
            pl.BlockSpec(memory_space=pltpu.SMEM),
        ],
        out_specs=pl.BlockSpec(memory_space=pltpu.VMEM),
        scratch_shapes=[
            pltpu.VMEM((2, CH, N), jnp.float32),
            pltpu.SemaphoreType.DMA((2,)),
            pltpu.SemaphoreType.DMA((2,)),
            pltpu.SemaphoreType.REGULAR,
        ],
        compiler_params=pltpu.CompilerParams(
            collective_id=0, vmem_limit_bytes=100 * 1024 * 1024),
    )(x, w_mat, scale_x, scale_w)


# device time: 481910 ns/iter; 1.6169x vs baseline; 1.6169x over previous
import jax
import jax.numpy as jnp
from jax import lax
from jax.experimental import pallas as pl
from jax.experimental.pallas import tpu as pltpu

N_DEV = 16
M, N = 4096, 2048
CH = M // N_DEV
NH = N // 2
N_STEPS = 2 * (N_DEV - 1)


def kernel(x, w_mat, scale_x, scale_w):
    def body(x_ref, w_ref, sx_ref, sw_ref, out_ref,
             comm_ref, send_sems, recv_sems, credit0_sem, credit1_sem):
        my = lax.axis_index("i")
        left = lax.rem(my + N_DEV - 1, N_DEV)
        right = lax.rem(my + 1, N_DEV)

        barrier_sem = pltpu.get_barrier_semaphore()
        for nbr in (left, right):
            pl.semaphore_signal(barrier_sem, inc=1, device_id=(nbr,),
                                device_id_type=pl.DeviceIdType.MESH)
        pl.semaphore_wait(barrier_sem, 2)

        def gemm(c, half):
            return jnp.dot(
                x_ref[pl.ds(c * CH, CH), :].astype(jnp.bfloat16),
                w_ref[:, half * NH:(half + 1) * NH].astype(jnp.bfloat16),
                preferred_element_type=jnp.float32,
            )

        scale = sx_ref[0] * sw_ref[0]

        comm_ref[0, 0, :, :] = gemm(my, 0)
        comm_ref[1, 0, :, :] = gemm(my, 1)

        for s in range(N_STEPS):
            send_slot = s % 2
            recv_slot = (s + 1) % 2
            if s >= 2:
                pl.semaphore_wait(credit0_sem, 1)
                pl.semaphore_wait(credit1_sem, 1)
            rdmas = []
            for r, dst in ((0, right), (1, left)):
                rdma = pltpu.make_async_remote_copy(
                    src_ref=comm_ref.at[r, send_slot],
                    dst_ref=comm_ref.at[r, recv_slot],
                    send_sem=send_sems.at[r, send_slot],
                    recv_sem=recv_sems.at[r, recv_slot],
                    device_id=(dst,),
                    device_id_type=pl.DeviceIdType.MESH,
                )
                rdma.start()
                rdmas.append(rdma)
            for rdma in rdmas:
                rdma.wait()

            if s < N_DEV - 1:
                cs = (lax.rem(my - s - 1 + N_DEV, N_DEV),
                      lax.rem(my + s + 1, N_DEV))
                for r, c in enumerate(cs):
                    acc = comm_ref[r, recv_slot, :, :] + gemm(c, r)
                    if s == N_DEV - 2:
                        acc = jnp.maximum(acc * scale, 0.0)
                        out_ref[pl.ds(c * CH, CH), r * NH:(r + 1) * NH] = acc
                    comm_ref[r, recv_slot, :, :] = acc
            else:
                cs = (lax.rem(my - (s - N_DEV + 1) + N_DEV, N_DEV),
                      lax.rem(my + (s - N_DEV + 1), N_DEV))
                for r, c in enumerate(cs):
                    out_ref[pl.ds(c * CH, CH), r * NH:(r + 1) * NH] = (
                        comm_ref[r, recv_slot, :, :])

            if s <= N_STEPS - 3:
                pl.semaphore_signal(credit0_sem, inc=1, device_id=(left,),
                                    device_id_type=pl.DeviceIdType.MESH)
                pl.semaphore_signal(credit1_sem, inc=1, device_id=(right,),
                                    device_id_type=pl.DeviceIdType.MESH)

    return pl.pallas_call(
        body,
        out_shape=jax.ShapeDtypeStruct((M, N), jnp.float32),
        in_specs=[
            pl.BlockSpec(memory_space=pltpu.VMEM),
            pl.BlockSpec(memory_space=pltpu.VMEM),
            pl.BlockSpec(memory_space=pltpu.SMEM),
            pl.BlockSpec(memory_space=pltpu.SMEM),
        ],
        out_specs=pl.BlockSpec(memory_space=pltpu.VMEM),
        scratch_shapes=[
            pltpu.VMEM((2, 2, CH, NH), jnp.float32),
            pltpu.SemaphoreType.DMA((2, 2)),
            pltpu.SemaphoreType.DMA((2, 2)),
            pltpu.SemaphoreType.REGULAR,
            pltpu.SemaphoreType.REGULAR,
        ],
        compiler_params=pltpu.CompilerParams(
            collective_id=0, vmem_limit_bytes=100 * 1024 * 1024),
    )(x, w_mat, scale_x, scale_w)


# device time: 214042 ns/iter; 3.6405x vs baseline; 2.2515x over previous
import jax
import jax.numpy as jnp
from jax import lax
from jax.experimental import pallas as pl
from jax.experimental.pallas import tpu as pltpu

N_DEV = 16
M, N = 4096, 2048
QR = M // 4
SR = QR // 4
AW = 640
BW = 384
MESH = pl.DeviceIdType.MESH


def kernel(x, w_mat, scale_x, scale_w):
    my = lax.axis_index("i")
    z = my // 4
    sg = lax.rem(my, 4)
    nxt_s = 4 * z + lax.rem(sg + 1, 4)
    prv_s = 4 * z + lax.rem(sg + 3, 4)
    nxt_z = 4 * lax.rem(z + 1, 4) + sg
    prv_z = 4 * lax.rem(z + 3, 4) + sg
    nbrs = jnp.stack([sg, z, nxt_s, prv_s, nxt_z, prv_z]).astype(jnp.int32)

    def body(x_ref, w_ref, sx_ref, sw_ref, nbr_ref, out_ref,
             commA, commB, commSA, commSB,
             sq_send, sq_recv, ss_send, ss_recv,
             cq0, cq1, cq2, cq3, cs0, cs1, cs2, cs3,
             pc0, pc1, pc2, pc3, eq0, eq1, eq2, eq3,
             es0, es1, es2, es3):
        sg = nbr_ref[0]
        z = nbr_ref[1]
        nxt_s = nbr_ref[2]
        prv_s = nbr_ref[3]
        nxt_z = nbr_ref[4]
        prv_z = nbr_ref[5]

        creditq = (cq0, cq1, cq2, cq3)
        credits = (cs0, cs1, cs2, cs3)
        phasec = (pc0, pc1, pc2, pc3)
        seedq = (eq0, eq1, eq2, eq3)
        seeds = (es0, es1, es2, es3)

        streams = (
            (0, AW, commA, 0, commSA,
             nxt_s, prv_s, sg, -1, nxt_z, prv_z, z, -1),
            (AW, AW, commA, 1, commSA,
             prv_s, nxt_s, sg, 1, prv_z, nxt_z, z, 1),
            (2 * AW, BW, commB, 0, commSB,
             nxt_z, prv_z, z, -1, nxt_s, prv_s, sg, -1),
            (2 * AW + BW, BW, commB, 1, commSB,
             prv_z, nxt_z, z, 1, prv_s, nxt_s, sg, 1),
        )

        def sget(st):
            (c0, w, q, d, sref, to1, frm1, pos1, walk1,
             to2, frm2, pos2, walk2) = st
            return (c0, w, q, d, sref, to1, frm1, pos1, walk1,
                    to2, frm2, pos2, walk2)

        barrier_sem = pltpu.get_barrier_semaphore()
        for nbr in (nxt_s, prv_s, nxt_z, prv_z):
            pl.semaphore_signal(barrier_sem, inc=1, device_id=(nbr,),
                                device_id_type=MESH)
        pl.semaphore_wait(barrier_sem, 4)

        def gemm(c, c0, w):
            return jnp.dot(
                x_ref[pl.ds(c * QR, QR), :].astype(jnp.bfloat16),
                w_ref[:, c0:c0 + w].astype(jnp.bfloat16),
                preferred_element_type=jnp.float32,
            )

        scale = sx_ref[0] * sw_ref[0]
        f32 = jnp.float32
        bf16 = jnp.bfloat16

        pending = {}

        def flush(group, k, slot):
            rdma = pending.pop((id(group), k, slot), None)
            if rdma is not None:
                rdma.wait_send()

        def send(group_send, group_recv, k, slot, src, dst, dev):
            flush(group_send, k, slot)
            rdma = pltpu.make_async_remote_copy(
                src_ref=src, dst_ref=dst,
                send_sem=group_send.at[k, slot],
                recv_sem=group_recv.at[k, slot],
                device_id=(dev,), device_id_type=MESH,
            )
            rdma.start()
            pending[(id(group_send), k, slot)] = rdma
            return rdma

        for st in streams:
            c0, w, q, d = st[0], st[1], st[2], st[3]
            pos1 = st[7]
            q[d, 0, :, :] = gemm(pos1, c0, w).astype(bf16)
        for s in range(3):
            snd, rcv = s % 2, (s + 1) % 2
            rdmas = []
            for k, st in enumerate(streams):
                q, d, to1 = st[2], st[3], st[5]
                if s == 2:
                    pl.semaphore_wait(creditq[k], 1)
                rdmas.append(send(sq_send, sq_recv, k, snd,
                                  q.at[d, snd], q.at[d, rcv], to1))
            for k, st in enumerate(streams):
                c0, w, q, d, frm1 = st[0], st[1], st[2], st[3], st[6]
                pos1, walk1 = st[7], st[8]
                rdmas[k].wait_recv()
                c = lax.rem(pos1 + walk1 * (s + 1) + 8, 4)
                acc = q[d, rcv, :, :].astype(f32) + gemm(c, c0, w)
                q[d, rcv, :, :] = acc.astype(bf16)
                if s == 0:
                    pl.semaphore_signal(creditq[k], inc=1,
                                        device_id=(frm1,), device_id_type=MESH)

        for st in streams:
            q, d, sref, pos2 = st[2], st[3], st[4], st[11]
            sref[d, 0, :, :] = q[d, 1, pl.ds(pos2 * SR, SR), :]
        for s in range(6):
            snd, rcv = s % 2, (s + 1) % 2
            rdmas = []
            for k, st in enumerate(streams):
                d, sref, to2 = st[3], st[4], st[9]
                if s == 1:
                    pl.semaphore_wait(seeds[k], 1)
                if s >= 2:
                    pl.semaphore_wait(credits[k], 1)
                rdmas.append(send(ss_send, ss_recv, k, snd,
                                  sref.at[d, snd], sref.at[d, rcv], to2))
            for k, st in enumerate(streams):
                c0, w, q, d, sref = st[0], st[1], st[2], st[3], st[4]
                frm1, pos1, walk1 = st[6], st[7], st[8]
                frm2, pos2, walk2 = st[10], st[11], st[12]
                o1 = lax.rem(pos1 - walk1 + 8, 4)
                rdmas[k].wait_recv()
                if s == 0:
                    flush(ss_send, k, 0)
                    pl.semaphore_signal(seeds[k], inc=1,
                                        device_id=(frm2,), device_id_type=MESH)
                if s < 3:
                    c = lax.rem(pos2 + walk2 * (s + 1) + 8, 4)
                    acc = (sref[d, rcv, :, :].astype(f32)
                           + q[d, 1, pl.ds(c * SR, SR), :].astype(f32))
                    if s == 2:
                        y = jnp.maximum(acc * scale, 0.0)
                        out_ref[pl.ds(o1 * QR + c * SR, SR), c0:c0 + w] = y
                        y16 = y.astype(bf16)
                        sref[d, rcv, :, :] = y16
                        q[d, 0, pl.ds(c * SR, SR), :] = y16
                        pl.semaphore_signal(phasec[k], inc=1,
                                            device_id=(frm1,),
                                            device_id_type=MESH)
                    else:
                        sref[d, rcv, :, :] = acc.astype(bf16)
                else:
                    c = lax.rem(pos2 + walk2 * (s - 3) + 8, 4)
                    y16 = sref[d, rcv, :, :]
                    out_ref[pl.ds(o1 * QR + c * SR, SR), c0:c0 + w] = (
                        y16.astype(f32))
                    q[d, 0, pl.ds(c * SR, SR), :] = y16
                if s <= 3:
                    pl.semaphore_signal(credits[k], inc=1,
                                        device_id=(frm2,), device_id_type=MESH)

        for k in range(4):
            pl.semaphore_wait(phasec[k], 1)
        for s in range(3):
            snd, rcv = s % 2, (s + 1) % 2
            rdmas = []
            for k, st in enumerate(streams):
                q, d, to1 = st[2], st[3], st[5]
                if s == 1:
                    pl.semaphore_wait(seedq[k], 1)
                if s == 2:
                    pl.semaphore_wait(creditq[k], 1)
                rdmas.append(send(sq_send, sq_recv, k, snd,
                                  q.at[d, snd], q.at[d, rcv], to1))
            for k, st in enumerate(streams):
                c0, w, q, d, frm1 = st[0], st[1], st[2], st[3], st[6]
                pos1, walk1 = st[7], st[8]
                rdmas[k].wait_recv()
                if s == 0:
                    flush(sq_send, k, 0)
                    pl.semaphore_signal(seedq[k], inc=1,
                                        device_id=(frm1,), device_id_type=MESH)
                c = lax.rem(pos1 + walk1 * s + 8, 4)
                out_ref[pl.ds(c * QR, QR), c0:c0 + w] = (
                    q[d, rcv, :, :].astype(f32))
                if s == 0:
                    pl.semaphore_signal(creditq[k], inc=1,
                                        device_id=(frm1,), device_id_type=MESH)

        for rdma in pending.values():
            rdma.wait_send()

    return pl.pallas_call(
        body,
        out_shape=jax.ShapeDtypeStruct((M, N), jnp.float32),
        in_specs=[
            pl.BlockSpec(memory_space=pltpu.VMEM),
            pl.BlockSpec(memory_space=pltpu.VMEM),
            pl.BlockSpec(memory_space=pltpu.SMEM),
            pl.BlockSpec(memory_space=pltpu.SMEM),
            pl.BlockSpec(memory_space=pltpu.SMEM),
        ],
        out_specs=pl.BlockSpec(memory_space=pltpu.VMEM),
        scratch_shapes=[
            pltpu.VMEM((2, 2, QR, AW), jnp.bfloat16),
            pltpu.VMEM((2, 2, QR, BW), jnp.bfloat16),
            pltpu.VMEM((2, 2, SR, AW), jnp.bfloat16),
            pltpu.VMEM((2, 2, SR, BW), jnp.bfloat16),
            pltpu.SemaphoreType.DMA((4, 2)),
            pltpu.SemaphoreType.DMA((4, 2)),
            pltpu.SemaphoreType.DMA((4, 2)),
            pltpu.SemaphoreType.DMA((4, 2)),
        ] + [pltpu.SemaphoreType.REGULAR] * 20,
        compiler_params=pltpu.CompilerParams(
            collective_id=0, vmem_limit_bytes=100 * 1024 * 1024),
    )(x, w_mat, scale_x, scale_w, nbrs)
